# Initial kernel scaffold; baseline (speedup 1.0000x reference)
import jax
import jax.numpy as jnp
from jax.experimental import pallas as pl


def kernel(x, edge_index, edge_attr, batch, params):
    raise NotImplementedError("write your pallas kernel here")


# speedup vs baseline: 3.8280x; 3.8280x over previous
import functools

import jax
import jax.numpy as jnp
from jax import lax
from jax.experimental import pallas as pl
from jax.experimental.pallas import tpu as pltpu
from jax.experimental.pallas import tpu_sc as plsc

N = 10000
E = 320000
D = 128
DE = 16

NC = 2
NS = 16
NW = NC * NS
EP = E // NW
B = 128
NFULL = EP // B
REM = EP - NFULL * B
NROWCH = N // B
ROWREM = N - NROWCH * B


def _linear_body(x_ref, w_ref, b_ref, o_ref, *, act):
    y = jnp.dot(x_ref[...], w_ref[...], preferred_element_type=jnp.float32)
    y = y + b_ref[...]
    if act:
        y = y * jax.nn.sigmoid(y)
    o_ref[...] = y


def _linear(x, w, b, act, block_rows):
    m, k = x.shape
    dout = w.shape[1]
    return pl.pallas_call(
        functools.partial(_linear_body, act=act),
        grid=(m // block_rows,),
        in_specs=[
            pl.BlockSpec((block_rows, k), lambda i: (i, 0)),
            pl.BlockSpec((k, dout), lambda i: (0, 0)),
            pl.BlockSpec((1, dout), lambda i: (0, 0)),
        ],
        out_specs=pl.BlockSpec((block_rows, dout), lambda i: (i, 0)),
        out_shape=jax.ShapeDtypeStruct((m, dout), jnp.float32),
    )(x, w, b.reshape(1, dout))


def _mlp_body(h_ref, p_ref, w1_ref, b1_ref, g1_ref, be1_ref, w2_ref, b2_ref,
              o_ref):
    t = h_ref[...] + p_ref[0] + p_ref[1]
    t = jnp.dot(t, w1_ref[...], preferred_element_type=jnp.float32)
    t = t + b1_ref[...]
    mu = jnp.mean(t, axis=-1, keepdims=True)
    var = jnp.mean((t - mu) ** 2, axis=-1, keepdims=True)
    t = (t - mu) / jnp.sqrt(var + 1e-5) * g1_ref[...] + be1_ref[...]
    t = t * jax.nn.sigmoid(t)
    t = jnp.dot(t, w2_ref[...], preferred_element_type=jnp.float32)
    t = t + b2_ref[...]
    o_ref[...] = t * jax.nn.sigmoid(t)


def _mlp(h, parts, lp, block_rows=2000):
    vec = lambda v: v.reshape(1, D)
    return pl.pallas_call(
        _mlp_body,
        grid=(N // block_rows,),
        in_specs=[
            pl.BlockSpec((block_rows, D), lambda i: (i, 0)),
            pl.BlockSpec((2, block_rows, D), lambda i: (0, i, 0)),
            pl.BlockSpec((D, D), lambda i: (0, 0)),
            pl.BlockSpec((1, D), lambda i: (0, 0)),
            pl.BlockSpec((1, D), lambda i: (0, 0)),
            pl.BlockSpec((1, D), lambda i: (0, 0)),
            pl.BlockSpec((D, D), lambda i: (0, 0)),
            pl.BlockSpec((1, D), lambda i: (0, 0)),
        ],
        out_specs=pl.BlockSpec((block_rows, D), lambda i: (i, 0)),
        out_shape=jax.ShapeDtypeStruct((N, D), jnp.float32),
    )(h, parts, lp['W1'], vec(lp['b1']), vec(lp['g1']), vec(lp['be1']),
      lp['W2'], vec(lp['b2']))


def _make_agg():
    mesh = plsc.VectorSubcoreMesh(core_axis_name="c", subcore_axis_name="s")

    @functools.partial(
        pl.kernel,
        mesh=mesh,
        out_type=jax.ShapeDtypeStruct((NC, N, D), jnp.float32),
        scratch_types=[
            pltpu.VMEM((B,), jnp.int32),
            pltpu.VMEM((B,), jnp.int32),
            pltpu.VMEM((B, D), jnp.float32),
            pltpu.VMEM((B, D), jnp.float32),
            pltpu.VMEM((REM,), jnp.int32),
            pltpu.VMEM((REM,), jnp.int32),
            pltpu.VMEM((REM, D), jnp.float32),
            pltpu.VMEM((REM, D), jnp.float32),
            pltpu.VMEM_SHARED((N, D), jnp.float32),
            pltpu.SemaphoreType.DMA,
        ],
    )
    def agg(h_hbm, e_hbm, src_hbm, dst_hbm, out_hbm,
            si, di, rows, ev, si2, di2, rows2, ev2, acc, sem):
        cid = lax.axis_index("c")
        sid = lax.axis_index("s")
        wid = sid * NC + cid

        zero16 = jnp.zeros((16,), jnp.float32)

        def zrow(i, _):
            for j in range(D // 16):
                rows[i, pl.ds(j * 16, 16)] = zero16
            return 0
        lax.fori_loop(0, B, zrow, 0)

        def zchunk(q, _):
            c = sid + q * NS

            @pl.when(c < NROWCH)
            def _():
                pltpu.sync_copy(rows, acc.at[pl.ds(c * B, B)])
            return 0
        lax.fori_loop(0, (NROWCH + NS - 1) // NS, zchunk, 0)

        @pl.when(sid == 0)
        def _():
            pltpu.sync_copy(rows.at[pl.ds(0, ROWREM)],
                            acc.at[pl.ds(NROWCH * B, ROWREM)])

        plsc.subcore_barrier()

        base = wid * EP

        def chunk(k, _):
            off = pl.multiple_of(base + k * B, 8)
            pltpu.sync_copy(src_hbm.at[pl.ds(off, B)], si)
            gather = pltpu.async_copy(h_hbm.at[si], rows, sem)
            pltpu.sync_copy(dst_hbm.at[pl.ds(off, B)], di)
            pltpu.sync_copy(e_hbm.at[pl.ds(off, B)], ev)
            gather.wait()

            def msg(i, _):
                for j in range(D // 16):
                    sl = pl.ds(j * 16, 16)
                    rows[i, sl] = jnp.maximum(rows[i, sl] + ev[i, sl], 0.0)
                return 0
            lax.fori_loop(0, B, msg, 0)

            pltpu.sync_copy(rows, acc.at[di], add=True)
            return 0
        lax.fori_loop(0, NFULL, chunk, 0)

        off = base + NFULL * B
        pltpu.sync_copy(src_hbm.at[pl.ds(off, REM)], si2)
        gather = pltpu.async_copy(h_hbm.at[si2], rows2, sem)
        pltpu.sync_copy(dst_hbm.at[pl.ds(off, REM)], di2)
        pltpu.sync_copy(e_hbm.at[pl.ds(off, REM)], ev2)
        gather.wait()

        def msg2(i, _):
            for j in range(D // 16):
                sl = pl.ds(j * 16, 16)
                rows2[i, sl] = jnp.maximum(rows2[i, sl] + ev2[i, sl], 0.0)
            return 0
        lax.fori_loop(0, REM, msg2, 0)

        pltpu.sync_copy(rows2, acc.at[di2], add=True)

        plsc.subcore_barrier()

        def ochunk(q, _):
            c = sid + q * NS

            @pl.when(c < NROWCH)
            def _():
                pltpu.sync_copy(acc.at[pl.ds(c * B, B)],
                                out_hbm.at[cid, pl.ds(c * B, B)])
            return 0
        lax.fori_loop(0, (NROWCH + NS - 1) // NS, ochunk, 0)

        @pl.when(sid == 0)
        def _():
            pltpu.sync_copy(acc.at[pl.ds(NROWCH * B, ROWREM)],
                            out_hbm.at[cid, pl.ds(NROWCH * B, ROWREM)])

    return agg


def kernel(x, edge_index, edge_attr, batch, params):
    p = params
    src = edge_index[0]
    dst = edge_index[1]

    h = _linear(x, p['Wa'], p['ba'], act=False, block_rows=2000)
    e = _linear(edge_attr, p['Wb'], p['bb'], act=False, block_rows=4000)

    agg = _make_agg()
    for lp in p['layers']:
        parts = agg(h, e, src, dst)
        h = _mlp(h, parts, lp)

    return _linear(h, p['Wl'], p['bl'], act=True, block_rows=2000)


# speedup vs baseline: 5.8838x; 1.5370x over previous
import functools

import jax
import jax.numpy as jnp
from jax import lax
from jax.experimental import pallas as pl
from jax.experimental.pallas import tpu as pltpu
from jax.experimental.pallas import tpu_sc as plsc

N = 10000
E = 320000
D = 128
DE = 16

NC = 2
NS = 16
NW = NC * NS
EP = E // NW
B = 64
NFULL = EP // B
REM = EP - NFULL * B
NROWCH = N // B
ROWREM = N - NROWCH * B


def _linear_body(x_ref, w_ref, b_ref, o_ref, *, act):
    y = jnp.dot(x_ref[...], w_ref[...], preferred_element_type=jnp.float32)
    y = y + b_ref[...]
    if act:
        y = y * jax.nn.sigmoid(y)
    o_ref[...] = y


def _linear(x, w, b, act, block_rows):
    m, k = x.shape
    dout = w.shape[1]
    return pl.pallas_call(
        functools.partial(_linear_body, act=act),
        grid=(m // block_rows,),
        in_specs=[
            pl.BlockSpec((block_rows, k), lambda i: (i, 0)),
            pl.BlockSpec((k, dout), lambda i: (0, 0)),
            pl.BlockSpec((1, dout), lambda i: (0, 0)),
        ],
        out_specs=pl.BlockSpec((block_rows, dout), lambda i: (i, 0)),
        out_shape=jax.ShapeDtypeStruct((m, dout), jnp.float32),
    )(x, w, b.reshape(1, dout))


def _mlp_body(h_ref, p_ref, w1_ref, b1_ref, g1_ref, be1_ref, w2_ref, b2_ref,
              o_ref):
    t = h_ref[...] + p_ref[0] + p_ref[1]
    t = jnp.dot(t, w1_ref[...], preferred_element_type=jnp.float32)
    t = t + b1_ref[...]
    mu = jnp.mean(t, axis=-1, keepdims=True)
    var = jnp.mean((t - mu) ** 2, axis=-1, keepdims=True)
    t = (t - mu) / jnp.sqrt(var + 1e-5) * g1_ref[...] + be1_ref[...]
    t = t * jax.nn.sigmoid(t)
    t = jnp.dot(t, w2_ref[...], preferred_element_type=jnp.float32)
    t = t + b2_ref[...]
    o_ref[...] = t * jax.nn.sigmoid(t)


def _mlp(h, parts, lp, block_rows=2000):
    vec = lambda v: v.reshape(1, D)
    return pl.pallas_call(
        _mlp_body,
        grid=(N // block_rows,),
        in_specs=[
            pl.BlockSpec((block_rows, D), lambda i: (i, 0)),
            pl.BlockSpec((2, block_rows, D), lambda i: (0, i, 0)),
            pl.BlockSpec((D, D), lambda i: (0, 0)),
            pl.BlockSpec((1, D), lambda i: (0, 0)),
            pl.BlockSpec((1, D), lambda i: (0, 0)),
            pl.BlockSpec((1, D), lambda i: (0, 0)),
            pl.BlockSpec((D, D), lambda i: (0, 0)),
            pl.BlockSpec((1, D), lambda i: (0, 0)),
        ],
        out_specs=pl.BlockSpec((block_rows, D), lambda i: (i, 0)),
        out_shape=jax.ShapeDtypeStruct((N, D), jnp.float32),
    )(h, parts, lp['W1'], vec(lp['b1']), vec(lp['g1']), vec(lp['be1']),
      lp['W2'], vec(lp['b2']))


def _make_agg():
    mesh = plsc.VectorSubcoreMesh(core_axis_name="c", subcore_axis_name="s")

    @functools.partial(
        pl.kernel,
        mesh=mesh,
        out_type=jax.ShapeDtypeStruct((NC, N, D), jnp.float32),
        scratch_types=[
            pltpu.VMEM((B,), jnp.int32),
            pltpu.VMEM((B,), jnp.int32),
            pltpu.VMEM((B,), jnp.int32),
            pltpu.VMEM((B,), jnp.int32),
            pltpu.VMEM((B,), jnp.int32),
            pltpu.VMEM((B,), jnp.int32),
            pltpu.VMEM((B, D), jnp.float32),
            pltpu.VMEM((B, D), jnp.float32),
            pltpu.VMEM((B, D), jnp.float32),
            pltpu.VMEM((B, D), jnp.float32),
            pltpu.VMEM((REM,), jnp.int32),
            pltpu.VMEM((REM,), jnp.int32),
            pltpu.VMEM((REM, D), jnp.float32),
            pltpu.VMEM((REM, D), jnp.float32),
            pltpu.VMEM_SHARED((N, D), jnp.float32),
            pltpu.SemaphoreType.DMA,
            pltpu.SemaphoreType.DMA,
            pltpu.SemaphoreType.DMA,
            pltpu.SemaphoreType.DMA,
            pltpu.SemaphoreType.DMA,
            pltpu.SemaphoreType.DMA,
            pltpu.SemaphoreType.DMA,
            pltpu.SemaphoreType.DMA,
            pltpu.SemaphoreType.DMA,
        ],
    )
    def agg(h_hbm, e_hbm, src_hbm, dst_hbm, out_hbm,
            si0, si1, di0, di1, dsc0, dsc1, rows0, rows1, ev0, ev1,
            si2, di2, rows2, ev2, acc,
            ssi0, ssi1, sde0, sde1, sg0, sg1, ssc0, ssc1, sem2):
        SI = (si0, si1)
        DI = (di0, di1)
        DSC = (dsc0, dsc1)
        ROWS = (rows0, rows1)
        EV = (ev0, ev1)
        SSI = (ssi0, ssi1)
        SDE = (sde0, sde1)
        SG = (sg0, sg1)
        SSC = (ssc0, ssc1)

        cid = lax.axis_index("c")
        sid = lax.axis_index("s")
        wid = sid * NC + cid
        base = wid * EP

        def prefetch(k, b):
            off = pl.multiple_of(base + k * B, 8)
            pltpu.async_copy(src_hbm.at[pl.ds(off, B)], SI[b], SSI[b])
            pltpu.async_copy(dst_hbm.at[pl.ds(off, B)], DI[b], SDE[b])
            pltpu.async_copy(e_hbm.at[pl.ds(off, B)], EV[b], SDE[b])

        def wait_si(b):
            pltpu.make_async_copy(src_hbm.at[pl.ds(0, B)], SI[b], SSI[b]).wait()

        def wait_de(b):
            pltpu.make_async_copy(dst_hbm.at[pl.ds(0, B)], DI[b], SDE[b]).wait()
            pltpu.make_async_copy(e_hbm.at[pl.ds(0, B)], EV[b], SDE[b]).wait()

        def wait_gather(b):
            pltpu.make_async_copy(h_hbm.at[SI[b]], ROWS[b], SG[b]).wait()

        def wait_scatter(b):
            pltpu.make_async_copy(ROWS[b], acc.at[DSC[b]], SSC[b]).wait()

        prefetch(0, 0)
        prefetch(1, 1)

        zero16 = jnp.zeros((16,), jnp.float32)

        @plsc.parallel_loop(0, B)
        def _(i):
            for j in range(D // 16):
                rows0[i, pl.ds(j * 16, 16)] = zero16

        def zchunk(q, _):
            c = sid + q * NS

            @pl.when(c < NROWCH)
            def _():
                pltpu.sync_copy(rows0, acc.at[pl.ds(c * B, B)])
            return 0
        lax.fori_loop(0, (NROWCH + NS - 1) // NS, zchunk, 0)

        @pl.when(sid == 0)
        def _():
            pltpu.sync_copy(rows0.at[pl.ds(0, ROWREM)],
                            acc.at[pl.ds(NROWCH * B, ROWREM)])

        plsc.subcore_barrier()

        wait_si(0)
        pltpu.async_copy(h_hbm.at[SI[0]], ROWS[0], SG[0])

        def chunk_body(k, q, b, first, gather_guard, pref_guard):
            b1 = 1 - b

            def issue_next():
                wait_si(b1)
                if not first:
                    wait_scatter(b1)
                pltpu.async_copy(h_hbm.at[SI[b1]], ROWS[b1], SG[b1])

            if gather_guard is None:
                issue_next()
            else:
                @pl.when(gather_guard)
                def _():
                    issue_next()

            wait_de(b)
            wait_gather(b)

            @plsc.parallel_loop(0, B, unroll=2)
            def _(i):
                for j in range(D // 16):
                    sl = pl.ds(j * 16, 16)
                    ROWS[b][i, sl] = jnp.maximum(
                        ROWS[b][i, sl] + EV[b][i, sl], 0.0)

            @plsc.parallel_loop(0, B, step=16)
            def _(i):
                DSC[b][pl.ds(i, 16)] = DI[b][pl.ds(i, 16)]

            pltpu.async_copy(ROWS[b], acc.at[DSC[b]], SSC[b], add=True)

            def issue_pref():
                prefetch(k + 2, b)

            if pref_guard is None:
                issue_pref()
            else:
                @pl.when(pref_guard)
                def _():
                    issue_pref()

        chunk_body(0, 0, 0, True, None, None)
        chunk_body(1, 0, 1, False, None, None)

        NPAIR = NFULL // 2

        def pair(q, _):
            k0 = q * 2
            tail = q < NPAIR - 1
            chunk_body(k0, q, 0, False, None, tail)
            chunk_body(k0 + 1, q, 1, False, tail, tail)
            return 0
        lax.fori_loop(1, NPAIR, pair, 0)

        wait_scatter(0)
        wait_scatter(1)

        off = base + NFULL * B
        pltpu.sync_copy(src_hbm.at[pl.ds(off, REM)], si2)
        gather = pltpu.async_copy(h_hbm.at[si2], rows2, sem2)
        pltpu.sync_copy(dst_hbm.at[pl.ds(off, REM)], di2)
        pltpu.sync_copy(e_hbm.at[pl.ds(off, REM)], ev2)
        gather.wait()

        def msg2(i, _):
            for j in range(D // 16):
                sl = pl.ds(j * 16, 16)
                rows2[i, sl] = jnp.maximum(rows2[i, sl] + ev2[i, sl], 0.0)
            return 0
        lax.fori_loop(0, REM, msg2, 0)

        pltpu.sync_copy(rows2, acc.at[di2], add=True)

        plsc.subcore_barrier()

        def ochunk(q, _):
            c = sid + q * NS

            @pl.when(c < NROWCH)
            def _():
                pltpu.sync_copy(acc.at[pl.ds(c * B, B)],
                                out_hbm.at[cid, pl.ds(c * B, B)])
            return 0
        lax.fori_loop(0, (NROWCH + NS - 1) // NS, ochunk, 0)

        @pl.when(sid == 0)
        def _():
            pltpu.sync_copy(acc.at[pl.ds(NROWCH * B, ROWREM)],
                            out_hbm.at[cid, pl.ds(NROWCH * B, ROWREM)])

    return agg


def kernel(x, edge_index, edge_attr, batch, params):
    p = params
    src = edge_index[0]
    dst = edge_index[1]

    h = _linear(x, p['Wa'], p['ba'], act=False, block_rows=2000)
    e = _linear(edge_attr, p['Wb'], p['bb'], act=False, block_rows=4000)

    agg = _make_agg()
    for lp in p['layers']:
        parts = agg(h, e, src, dst)
        h = _mlp(h, parts, lp)

    return _linear(h, p['Wl'], p['bl'], act=True, block_rows=2000)
